# baseline (speedup 1.0000x reference)
import functools

import jax
import jax.numpy as jnp
from jax import lax
from jax.experimental import pallas as pl
from jax.experimental.pallas import tpu as pltpu
from jax.experimental.pallas import tpu_sc as plsc

VOCAB = 256
HIDDEN = 5
CTX = 8
HPAD = 16
FPAD = CTX * HPAD
NW = 32
NCH = 4
LANES = 16


@functools.lru_cache(maxsize=None)
def _make_sc_gather(batch: int, ctx: int):
    rows_w = batch // NW
    rows_ch = rows_w // NCH
    mesh = plsc.VectorSubcoreMesh(core_axis_name="c", subcore_axis_name="s")

    @functools.partial(
        pl.kernel,
        out_type=jax.ShapeDtypeStruct((batch, FPAD), jnp.float32),
        mesh=mesh,
        scratch_types=[
            pltpu.VMEM((rows_w * ctx // 128, 128), jnp.int32),
            pltpu.VMEM((VOCAB, HPAD), jnp.float32),
            pltpu.VMEM((rows_w, FPAD), jnp.float32),
            pltpu.SemaphoreType.DMA,
        ],
        compiler_params=pltpu.CompilerParams(
            use_tc_tiling_on_sc=False, needs_layout_passes=False),
    )
    def sc_gather(idx_hbm, table_hbm, out_hbm, idx_v, table_v, out_v, wsem):
        wid = lax.axis_index("s") * 2 + lax.axis_index("c")
        base = wid * rows_w
        pltpu.sync_copy(table_hbm, table_v)
        pltpu.sync_copy(
            idx_hbm.at[pl.ds(base * ctx // 128, rows_w * ctx // 128)], idx_v)
        col = lax.iota(jnp.int32, LANES)
        writes = []
        for cb in range(NCH):
            @functools.partial(
                plsc.parallel_loop,
                cb * rows_ch * ctx, (cb + 1) * rows_ch * ctx, unroll=8)
            def body(o):
                r = lax.shift_right_logical(o, 3)
                c = lax.bitwise_and(o, 7)
                rows = plsc.load_gather(
                    idx_v, [lax.broadcast(lax.shift_right_logical(o, 7), (LANES,)),
                            lax.broadcast(lax.bitwise_and(o, 127), (LANES,))])
                vals = plsc.load_gather(table_v, [rows, col])
                out_v[r, pl.ds(c * HPAD, HPAD)] = vals
            writes.append(pltpu.async_copy(
                out_v.at[pl.ds(cb * rows_ch, rows_ch)],
                out_hbm.at[pl.ds(base + cb * rows_ch, rows_ch)],
                wsem))
        for w in writes:
            w.wait()

    return sc_gather


def _dense_body(emb_ref, w_ref, b_ref, out_ref):
    out_ref[...] = lax.dot_general(
        emb_ref[...], w_ref[...], (((1,), (1,)), ((), ())),
        preferred_element_type=jnp.float32) + b_ref[...]


def _dense(emb, w_pad, b2d, batch: int, tile: int):
    grid = (batch // tile,)
    return pl.pallas_call(
        _dense_body,
        grid=grid,
        in_specs=[
            pl.BlockSpec((tile, FPAD), lambda i: (i, 0)),
            pl.BlockSpec((VOCAB, FPAD), lambda i: (0, 0)),
            pl.BlockSpec((1, VOCAB), lambda i: (0, 0)),
        ],
        out_specs=pl.BlockSpec((tile, VOCAB), lambda i: (i, 0)),
        out_shape=jax.ShapeDtypeStruct((batch, VOCAB), jnp.float32),
    )(emb, w_pad, b2d)


def kernel(x, embed_table, fc_w, fc_b):
    batch, ctx = x.shape
    vocab, hidden = embed_table.shape

    table_pad = jnp.pad(embed_table, ((0, 0), (0, HPAD - hidden)))
    w_pad = jnp.pad(
        fc_w.reshape(vocab, ctx, hidden), ((0, 0), (0, 0), (0, HPAD - hidden))
    ).reshape(vocab, ctx * HPAD)

    idx2d = x.reshape(batch * ctx // 128, 128)
    emb = _make_sc_gather(batch, ctx)(idx2d, table_pad)
    return _dense(emb, w_pad, fc_b.reshape(1, vocab), batch, tile=2048)


# speedup vs baseline: 1.0308x; 1.0308x over previous
import functools

import jax
import jax.numpy as jnp
from jax import lax
from jax.experimental import pallas as pl
from jax.experimental.pallas import tpu as pltpu
from jax.experimental.pallas import tpu_sc as plsc

VOCAB = 256
HIDDEN = 5
CTX = 8
HPAD = 16
FPAD = CTX * HPAD
NW = 32
NCH = 4
LANES = 16


@functools.lru_cache(maxsize=None)
def _make_sc_gather(batch: int, ctx: int):
    rows_w = batch // NW
    rows_ch = rows_w // NCH
    mesh = plsc.VectorSubcoreMesh(core_axis_name="c", subcore_axis_name="s")

    @functools.partial(
        pl.kernel,
        out_type=jax.ShapeDtypeStruct((batch, FPAD), jnp.float32),
        mesh=mesh,
        scratch_types=[
            pltpu.VMEM((rows_w * ctx // 128, 128), jnp.int32),
            pltpu.VMEM((VOCAB, HPAD), jnp.float32),
            pltpu.VMEM((rows_w, FPAD), jnp.float32),
            pltpu.SemaphoreType.DMA,
        ],
        compiler_params=pltpu.CompilerParams(
            use_tc_tiling_on_sc=False, needs_layout_passes=False),
    )
    def sc_gather(idx_hbm, table_hbm, out_hbm, idx_v, table_v, out_v, wsem):
        wid = lax.axis_index("s") * 2 + lax.axis_index("c")
        base = wid * rows_w
        pltpu.sync_copy(table_hbm, table_v)
        pltpu.sync_copy(
            idx_hbm.at[pl.ds(base * ctx // 128, rows_w * ctx // 128)], idx_v)
        col = lax.iota(jnp.int32, LANES)
        writes = []
        for cb in range(NCH):
            @functools.partial(
                plsc.parallel_loop,
                cb * rows_ch * ctx, (cb + 1) * rows_ch * ctx, unroll=8)
            def body(o):
                r = lax.shift_right_logical(o, 3)
                c = lax.bitwise_and(o, 7)
                rows = plsc.load_gather(
                    idx_v, [lax.broadcast(lax.shift_right_logical(o, 7), (LANES,)),
                            lax.broadcast(lax.bitwise_and(o, 127), (LANES,))])
                vals = plsc.load_gather(table_v, [rows, col])
                out_v[r, pl.ds(c * HPAD, HPAD)] = vals
            writes.append(pltpu.async_copy(
                out_v.at[pl.ds(cb * rows_ch, rows_ch)],
                out_hbm.at[pl.ds(base + cb * rows_ch, rows_ch)],
                wsem))
        for w in writes:
            w.wait()

    return sc_gather


def _dense_body(emb_ref, w_ref, b_ref, out_ref):
    out_ref[...] = lax.dot_general(
        emb_ref[...], w_ref[...], (((1,), (1,)), ((), ())),
        preferred_element_type=jnp.float32) + b_ref[...]


def _dense(emb, w_pad, b2d, batch: int, tile: int):
    grid = (batch // tile,)
    return pl.pallas_call(
        _dense_body,
        grid=grid,
        in_specs=[
            pl.BlockSpec((tile, FPAD), lambda i: (i, 0)),
            pl.BlockSpec((VOCAB, FPAD), lambda i: (0, 0)),
            pl.BlockSpec((1, VOCAB), lambda i: (0, 0)),
        ],
        out_specs=pl.BlockSpec((tile, VOCAB), lambda i: (i, 0)),
        out_shape=jax.ShapeDtypeStruct((batch, VOCAB), jnp.float32),
    )(emb, w_pad, b2d)


def kernel(x, embed_table, fc_w, fc_b):
    batch, ctx = x.shape
    vocab, hidden = embed_table.shape

    table_pad = jnp.pad(embed_table, ((0, 0), (0, HPAD - hidden)))
    w_pad = jnp.pad(
        fc_w.reshape(vocab, ctx, hidden), ((0, 0), (0, 0), (0, HPAD - hidden))
    ).reshape(vocab, ctx * HPAD)

    idx2d = x.reshape(batch * ctx // 128, 128)
    emb = _make_sc_gather(batch, ctx)(idx2d, table_pad)
    return _dense(emb, w_pad, fc_b.reshape(1, vocab), batch, tile=4096)


# speedup vs baseline: 1.0658x; 1.0339x over previous
import functools

import jax
import jax.numpy as jnp
from jax import lax
from jax.experimental import pallas as pl
from jax.experimental.pallas import tpu as pltpu
from jax.experimental.pallas import tpu_sc as plsc

VOCAB = 256
HIDDEN = 5
CTX = 8
HPAD = 16
FPAD = CTX * HPAD
NW = 32
NCH = 4
LANES = 16


@functools.lru_cache(maxsize=None)
def _make_sc_gather(batch: int, ctx: int):
    rows_w = batch // NW
    rows_ch = rows_w // NCH
    mesh = plsc.VectorSubcoreMesh(core_axis_name="c", subcore_axis_name="s")

    @functools.partial(
        pl.kernel,
        out_type=jax.ShapeDtypeStruct((batch, FPAD), jnp.float32),
        mesh=mesh,
        scratch_types=[
            pltpu.VMEM((rows_w, CTX), jnp.int32),
            pltpu.VMEM((VOCAB, HPAD), jnp.float32),
            pltpu.VMEM((rows_w, FPAD), jnp.float32),
            pltpu.SemaphoreType.DMA,
        ],
        compiler_params=pltpu.CompilerParams(
            use_tc_tiling_on_sc=False, needs_layout_passes=False),
    )
    def sc_gather(idx_hbm, table_hbm, out_hbm, idx_v, table_v, out_v, wsem):
        wid = lax.axis_index("s") * 2 + lax.axis_index("c")
        base = wid * rows_w
        pltpu.sync_copy(table_hbm, table_v)
        pltpu.sync_copy(idx_hbm.at[pl.ds(base, rows_w), pl.ds(0, CTX)], idx_v)
        col = lax.iota(jnp.int32, LANES)
        writes = []
        for cb in range(NCH):
            @functools.partial(
                plsc.parallel_loop,
                cb * rows_ch * ctx, (cb + 1) * rows_ch * ctx, unroll=8)
            def body(o):
                r = lax.shift_right_logical(o, 3)
                c = lax.bitwise_and(o, 7)
                rows = plsc.load_gather(
                    idx_v, [lax.broadcast(r, (LANES,)),
                            lax.broadcast(c, (LANES,))])
                vals = plsc.load_gather(table_v, [rows, col])
                out_v[r, pl.ds(c * HPAD, HPAD)] = vals
            writes.append(pltpu.async_copy(
                out_v.at[pl.ds(cb * rows_ch, rows_ch)],
                out_hbm.at[pl.ds(base + cb * rows_ch, rows_ch)],
                wsem))
        for w in writes:
            w.wait()

    return sc_gather


def _dense_body(emb_ref, w_ref, b_ref, out_ref):
    out_ref[...] = lax.dot_general(
        emb_ref[...], w_ref[...], (((1,), (1,)), ((), ())),
        preferred_element_type=jnp.float32) + b_ref[...]


def _dense(emb, w_pad, b2d, batch: int, tile: int):
    grid = (batch // tile,)
    return pl.pallas_call(
        _dense_body,
        grid=grid,
        in_specs=[
            pl.BlockSpec((tile, FPAD), lambda i: (i, 0)),
            pl.BlockSpec((VOCAB, FPAD), lambda i: (0, 0)),
            pl.BlockSpec((1, VOCAB), lambda i: (0, 0)),
        ],
        out_specs=pl.BlockSpec((tile, VOCAB), lambda i: (i, 0)),
        out_shape=jax.ShapeDtypeStruct((batch, VOCAB), jnp.float32),
    )(emb, w_pad, b2d)


def kernel(x, embed_table, fc_w, fc_b):
    batch, ctx = x.shape
    vocab, hidden = embed_table.shape

    table_pad = jnp.pad(embed_table, ((0, 0), (0, HPAD - hidden)))
    w_pad = jnp.pad(
        fc_w.reshape(vocab, ctx, hidden), ((0, 0), (0, 0), (0, HPAD - hidden))
    ).reshape(vocab, ctx * HPAD)

    idx2d = jnp.pad(x, ((0, 0), (0, 128 - ctx)))
    emb = _make_sc_gather(batch, ctx)(idx2d, table_pad)
    return _dense(emb, w_pad, fc_b.reshape(1, vocab), batch, tile=4096)


# speedup vs baseline: 1.0664x; 1.0006x over previous
import functools

import jax
import jax.numpy as jnp
from jax import lax
from jax.experimental import pallas as pl
from jax.experimental.pallas import tpu as pltpu
from jax.experimental.pallas import tpu_sc as plsc

VOCAB = 256
HIDDEN = 5
CTX = 8
HPAD = 16
FPAD = CTX * HPAD
NW = 32
NCH = 2
LANES = 16


@functools.lru_cache(maxsize=None)
def _make_sc_gather(batch: int, ctx: int):
    rows_w = batch // NW
    rows_ch = rows_w // NCH
    mesh = plsc.VectorSubcoreMesh(core_axis_name="c", subcore_axis_name="s")

    @functools.partial(
        pl.kernel,
        out_type=jax.ShapeDtypeStruct((batch, FPAD), jnp.float32),
        mesh=mesh,
        scratch_types=[
            pltpu.VMEM((rows_w, CTX), jnp.int32),
            pltpu.VMEM((VOCAB, HPAD), jnp.float32),
            pltpu.VMEM((rows_w, FPAD), jnp.float32),
            pltpu.SemaphoreType.DMA,
        ],
        compiler_params=pltpu.CompilerParams(
            use_tc_tiling_on_sc=False, needs_layout_passes=False),
    )
    def sc_gather(idx_hbm, table_hbm, out_hbm, idx_v, table_v, out_v, wsem):
        wid = lax.axis_index("s") * 2 + lax.axis_index("c")
        base = wid * rows_w
        pltpu.sync_copy(table_hbm, table_v)
        pltpu.sync_copy(idx_hbm.at[pl.ds(base, rows_w), pl.ds(0, CTX)], idx_v)
        col = lax.iota(jnp.int32, LANES)
        writes = []
        for cb in range(NCH):
            @functools.partial(
                plsc.parallel_loop,
                cb * rows_ch * ctx, (cb + 1) * rows_ch * ctx, unroll=8)
            def body(o):
                r = lax.shift_right_logical(o, 3)
                c = lax.bitwise_and(o, 7)
                rows = plsc.load_gather(
                    idx_v, [lax.broadcast(r, (LANES,)),
                            lax.broadcast(c, (LANES,))])
                vals = plsc.load_gather(table_v, [rows, col])
                out_v[r, pl.ds(c * HPAD, HPAD)] = vals
            writes.append(pltpu.async_copy(
                out_v.at[pl.ds(cb * rows_ch, rows_ch)],
                out_hbm.at[pl.ds(base + cb * rows_ch, rows_ch)],
                wsem))
        for w in writes:
            w.wait()

    return sc_gather


def _dense_body(emb_ref, w_ref, b_ref, out_ref):
    out_ref[...] = lax.dot_general(
        emb_ref[...], w_ref[...], (((1,), (1,)), ((), ())),
        preferred_element_type=jnp.float32) + b_ref[...]


def _dense(emb, w_pad, b2d, batch: int, tile: int):
    grid = (batch // tile,)
    return pl.pallas_call(
        _dense_body,
        grid=grid,
        in_specs=[
            pl.BlockSpec((tile, FPAD), lambda i: (i, 0)),
            pl.BlockSpec((VOCAB, FPAD), lambda i: (0, 0)),
            pl.BlockSpec((1, VOCAB), lambda i: (0, 0)),
        ],
        out_specs=pl.BlockSpec((tile, VOCAB), lambda i: (i, 0)),
        out_shape=jax.ShapeDtypeStruct((batch, VOCAB), jnp.float32),
    )(emb, w_pad, b2d)


def kernel(x, embed_table, fc_w, fc_b):
    batch, ctx = x.shape
    vocab, hidden = embed_table.shape

    table_pad = jnp.pad(embed_table, ((0, 0), (0, HPAD - hidden)))
    w_pad = jnp.pad(
        fc_w.reshape(vocab, ctx, hidden), ((0, 0), (0, 0), (0, HPAD - hidden))
    ).reshape(vocab, ctx * HPAD)

    idx2d = jnp.pad(x, ((0, 0), (0, 128 - ctx)))
    emb = _make_sc_gather(batch, ctx)(idx2d, table_pad)
    return _dense(emb, w_pad, fc_b.reshape(1, vocab), batch, tile=4096)
